# baseline (speedup 1.0000x reference)
import functools

import numpy as np
import jax
import jax.numpy as jnp
from jax import lax
from jax.experimental import pallas as pl
from jax.experimental.pallas import tpu as pltpu
from jax.experimental.pallas import tpu_sc as plsc

N = 4096
NPAIRS = N * (N - 1) // 2
FLAT = 2 * NPAIRS
NB = NPAIRS // 128
NC, NS, L = 2, 16, 16
NW_ACT = 30
CPW = 13
NCHUNKS = NW_ACT * CPW
CBLOCKS = NB // NCHUNKS
CPAIRS = CBLOCKS * 128
CHUNK = 2 * CPAIRS
RECW = L
MAXIT = 224
XPAD = N + L
BUFPAD = 256 + L


def _build_schedule():
    i = np.arange(N, dtype=np.int64)
    off = i * (N - 1) - i * (i - 1) // 2
    chunk_items = []
    for g in range(NCHUNKS):
        p0, p1 = g * CPAIRS, (g + 1) * CPAIRS
        lo = int(np.searchsorted(off, p0, side="right")) - 1
        hi = int(np.searchsorted(off, p1, side="left"))
        items = []
        for si in range(lo, hi):
            s0 = max(int(off[si]), p0)
            s1 = min(int(off[si]) + (N - 1 - si), p1)
            if s1 <= s0:
                continue
            items.append((si, si + 1 + (s0 - int(off[si])), s0 - p0, s1 - p0))
        assert len(items) <= MAXIT, len(items)
        chunk_items.append(items)
    order = sorted(range(NCHUNKS), key=lambda g: -len(chunk_items[g]))
    loads = [0] * NW_ACT
    counts = [0] * NW_ACT
    assign = [[] for _ in range(NW_ACT)]
    for g in order:
        w = min((u for u in range(NW_ACT) if counts[u] < CPW), key=lambda u: loads[u])
        assign[w].append(g)
        counts[w] += 1
        loads[w] += CPAIRS // L + 40 * len(chunk_items[g])
    headers = np.zeros((NW_ACT * L, L), dtype=np.int32)
    recs = []
    for w in range(NW_ACT):
        for c, g in enumerate(assign[w]):
            items = chunk_items[g]
            headers[w * L + c, :3] = (len(items), g * CHUNK, len(recs) * RECW)
            for (si, j0, q0, q1) in items:
                rec = np.zeros(RECW, dtype=np.int32)
                rec[0] = q0
                rec[1] = q0 + ((q1 - q0) // L) * L
                rec[2] = si
                rec[3] = j0 - q0
                rec[4] = q1
                recs.append(rec)
    pad = [np.zeros(RECW, dtype=np.int32)] * MAXIT
    return headers.reshape(-1), np.stack(recs + pad).reshape(-1)


_HEADERS, _ITEMS = _build_schedule()


@functools.cache
def _get_pairs_kernel():
    mesh = plsc.VectorSubcoreMesh(
        core_axis_name="c", subcore_axis_name="s", num_cores=NC, num_subcores=NS
    )
    return functools.partial(
        pl.kernel,
        out_type=jax.ShapeDtypeStruct((FLAT,), jnp.float32),
        mesh=mesh,
        scratch_types=[
            pltpu.VMEM((XPAD,), jnp.float32),
            pltpu.VMEM((CPW * L,), jnp.int32),
            pltpu.VMEM((MAXIT * RECW,), jnp.int32),
            pltpu.VMEM((MAXIT * RECW,), jnp.int32),
            pltpu.VMEM((CHUNK + BUFPAD,), jnp.float32),
            pltpu.VMEM((CHUNK + BUFPAD,), jnp.float32),
            pltpu.SemaphoreType.DMA,
            pltpu.SemaphoreType.DMA,
            pltpu.SemaphoreType.DMA,
            pltpu.SemaphoreType.DMA,
        ],
        compiler_params=pltpu.CompilerParams(needs_layout_passes=False),
    )(_pairs_body)


def _pairs_body(x_hbm, hdr_hbm, items_hbm, out_hbm,
                xv, hdrv, itv0, itv1, buf0, buf1,
                osem0, osem1, isem0, isem1):
    w = lax.axis_index("s") * NC + lax.axis_index("c")

    @pl.when(w < NW_ACT)
    def _run():
        pltpu.sync_copy(x_hbm, xv.at[pl.ds(0, N)])
        pltpu.sync_copy(hdr_hbm.at[pl.ds(w * (L * L), CPW * L)], hdrv)
        iota = lax.iota(jnp.int32, L)
        bufs = (buf0, buf1)
        itvs = (itv0, itv1)
        osems = (osem0, osem1)
        isems = (isem0, isem1)

        def hdr_fields(c):
            hv = hdrv[pl.ds(c * L, L)]
            return hv[0], pl.multiple_of(hv[1], 8), pl.multiple_of(hv[2], 8)

        def prefetch(c):
            _, _, it_off = hdr_fields(c)
            return pltpu.async_copy(
                items_hbm.at[pl.ds(it_off, MAXIT * RECW)],
                itvs[c & 1],
                isems[c & 1],
            )

        out_descs = [None, None]
        it_descs = [None, None]
        it_descs[0] = prefetch(0)
        for c in range(CPW):
            nit, ooff, _ = hdr_fields(c)
            buf = bufs[c & 1]
            itv = itvs[c & 1]
            it_descs[c & 1].wait()
            if c + 1 < CPW:
                it_descs[(c + 1) & 1] = prefetch(c + 1)
            if out_descs[c & 1] is not None:
                out_descs[c & 1].wait()

            def item_body(k, carry, buf=buf, itv=itv):
                rec = itv[pl.ds(k * RECW, L)]
                q0 = rec[0]
                fe = rec[1]
                q1 = rec[4]
                xi = plsc.load_gather(xv, [jnp.full((L,), rec[2])])
                djv = jnp.full((L,), rec[3])
                pv0 = q0 + iota

                @plsc.parallel_loop(q0, fe, step=L, unroll=16,
                                    carry=(pv0, pv0 + djv))
                def _vec(t, st):
                    pv, jidx = st
                    pos = pv + (pv & -128)
                    jval = plsc.load_gather(xv, [jidx])
                    plsc.store_scatter(buf, [pos], xi)
                    plsc.store_scatter(buf, [pos + 128], jval)
                    return (pv + L, jidx + L)

                @pl.when(fe < q1)
                def _tail():
                    pv = fe + iota
                    pos = pv + (pv & -128)
                    mask = pv < jnp.full((L,), q1)
                    jval = plsc.load_gather(xv, [pv + djv])
                    plsc.store_scatter(buf, [pos], xi, mask=mask)
                    plsc.store_scatter(buf, [pos + 128], jval, mask=mask)

                return carry

            lax.fori_loop(0, nit, item_body, jnp.int32(0))
            out_descs[c & 1] = pltpu.async_copy(
                buf.at[pl.ds(0, CHUNK)], out_hbm.at[pl.ds(ooff, CHUNK)],
                osems[c & 1],
            )
        out_descs[0].wait()
        out_descs[1].wait()


def kernel(x):
    flat = _get_pairs_kernel()(
        x.reshape(-1), jnp.asarray(_HEADERS), jnp.asarray(_ITEMS)
    )
    return flat.reshape(NB, 2, 128).swapaxes(1, 2).reshape(NPAIRS, 2)


# speedup vs baseline: 1.1244x; 1.1244x over previous
import functools

import numpy as np
import jax
import jax.numpy as jnp
from jax import lax
from jax.experimental import pallas as pl
from jax.experimental.pallas import tpu as pltpu
from jax.experimental.pallas import tpu_sc as plsc

N = 4096
NPAIRS = N * (N - 1) // 2
FLAT = 2 * NPAIRS
NB = NPAIRS // 128
NC, NS, L = 2, 16, 16
NW_ACT = 32
CPW = 16
NCHUNKS = NW_ACT * CPW
CBLOCKS = 128
CPAIRS = CBLOCKS * 128
CHUNK = 2 * CPAIRS
RECW = L
MAXIT = 192
XPAD = N + L
BUFPAD = 256 + L


def _build_schedule():
    i = np.arange(N, dtype=np.int64)
    off = i * (N - 1) - i * (i - 1) // 2
    starts = [min(g * CPAIRS, NPAIRS - CPAIRS) for g in range(NCHUNKS)]
    chunk_items = []
    for g in range(NCHUNKS):
        p0, p1 = starts[g], starts[g] + CPAIRS
        lo = int(np.searchsorted(off, p0, side="right")) - 1
        hi = int(np.searchsorted(off, p1, side="left"))
        items = []
        for si in range(lo, hi):
            s0 = max(int(off[si]), p0)
            s1 = min(int(off[si]) + (N - 1 - si), p1)
            if s1 <= s0:
                continue
            items.append((si, si + 1 + (s0 - int(off[si])), s0 - p0, s1 - p0))
        assert len(items) <= MAXIT, len(items)
        chunk_items.append(items)
    order = sorted(range(NCHUNKS), key=lambda g: -len(chunk_items[g]))
    loads = [0] * NW_ACT
    counts = [0] * NW_ACT
    assign = [[] for _ in range(NW_ACT)]
    for g in order:
        w = min((u for u in range(NW_ACT) if counts[u] < CPW), key=lambda u: loads[u])
        assign[w].append(g)
        counts[w] += 1
        loads[w] += CPAIRS // L + 40 * len(chunk_items[g])
    headers = np.zeros((NW_ACT * L, L), dtype=np.int32)
    recs = []
    for w in range(NW_ACT):
        for c, g in enumerate(assign[w]):
            items = chunk_items[g]
            headers[w * L + c, :3] = (len(items), starts[g] * 2, len(recs) * RECW)
            for (si, j0, q0, q1) in items:
                rec = np.zeros(RECW, dtype=np.int32)
                rec[0] = q0
                rec[1] = q0 + ((q1 - q0) // L) * L
                rec[2] = si
                rec[3] = j0 - q0
                rec[4] = q1
                recs.append(rec)
    pad = [np.zeros(RECW, dtype=np.int32)] * MAXIT
    return headers.reshape(-1), np.stack(recs + pad).reshape(-1)


_HEADERS, _ITEMS = _build_schedule()


@functools.cache
def _get_pairs_kernel():
    mesh = plsc.VectorSubcoreMesh(
        core_axis_name="c", subcore_axis_name="s", num_cores=NC, num_subcores=NS
    )
    return functools.partial(
        pl.kernel,
        out_type=jax.ShapeDtypeStruct((FLAT,), jnp.float32),
        mesh=mesh,
        scratch_types=[
            pltpu.VMEM((XPAD,), jnp.float32),
            pltpu.VMEM((CPW * L,), jnp.int32),
            pltpu.VMEM((MAXIT * RECW,), jnp.int32),
            pltpu.VMEM((MAXIT * RECW,), jnp.int32),
            pltpu.VMEM((CHUNK + BUFPAD,), jnp.float32),
            pltpu.VMEM((CHUNK + BUFPAD,), jnp.float32),
            pltpu.SemaphoreType.DMA,
            pltpu.SemaphoreType.DMA,
            pltpu.SemaphoreType.DMA,
            pltpu.SemaphoreType.DMA,
        ],
        compiler_params=pltpu.CompilerParams(needs_layout_passes=False),
    )(_pairs_body)


def _pairs_body(x_hbm, hdr_hbm, items_hbm, out_hbm,
                xv, hdrv, itv0, itv1, buf0, buf1,
                osem0, osem1, isem0, isem1):
    w = lax.axis_index("s") * NC + lax.axis_index("c")

    @pl.when(w < NW_ACT)
    def _run():
        pltpu.sync_copy(x_hbm, xv.at[pl.ds(0, N)])
        pltpu.sync_copy(hdr_hbm.at[pl.ds(w * (L * L), CPW * L)], hdrv)
        iota = lax.iota(jnp.int32, L)
        bufs = (buf0, buf1)
        itvs = (itv0, itv1)
        osems = (osem0, osem1)
        isems = (isem0, isem1)

        def hdr_fields(c):
            hv = hdrv[pl.ds(c * L, L)]
            return hv[0], pl.multiple_of(hv[1], 8), pl.multiple_of(hv[2], 8)

        def prefetch(c):
            _, _, it_off = hdr_fields(c)
            return pltpu.async_copy(
                items_hbm.at[pl.ds(it_off, MAXIT * RECW)],
                itvs[c & 1],
                isems[c & 1],
            )

        out_descs = [None, None]
        it_descs = [None, None]
        it_descs[0] = prefetch(0)
        for c in range(CPW):
            nit, ooff, _ = hdr_fields(c)
            buf = bufs[c & 1]
            itv = itvs[c & 1]
            it_descs[c & 1].wait()
            if c + 1 < CPW:
                it_descs[(c + 1) & 1] = prefetch(c + 1)
            if out_descs[c & 1] is not None:
                out_descs[c & 1].wait()

            def item_body(k, carry, buf=buf, itv=itv):
                rec = itv[pl.ds(k * RECW, L)]
                q0 = rec[0]
                fe = rec[1]
                q1 = rec[4]
                xi = plsc.load_gather(xv, [jnp.full((L,), rec[2])])
                djv = jnp.full((L,), rec[3])
                pv0 = q0 + iota

                @plsc.parallel_loop(q0, fe, step=L, unroll=8,
                                    carry=(pv0, pv0 + djv))
                def _vec(t, st):
                    pv, jidx = st
                    pos = pv + (pv & -128)
                    jval = plsc.load_gather(xv, [jidx])
                    plsc.store_scatter(buf, [pos], xi)
                    plsc.store_scatter(buf, [pos + 128], jval)
                    return (pv + L, jidx + L)

                @pl.when(fe < q1)
                def _tail():
                    pv = fe + iota
                    pos = pv + (pv & -128)
                    mask = pv < jnp.full((L,), q1)
                    jval = plsc.load_gather(xv, [pv + djv])
                    plsc.store_scatter(buf, [pos], xi, mask=mask)
                    plsc.store_scatter(buf, [pos + 128], jval, mask=mask)

                return carry

            lax.fori_loop(0, nit, item_body, jnp.int32(0))
            out_descs[c & 1] = pltpu.async_copy(
                buf.at[pl.ds(0, CHUNK)], out_hbm.at[pl.ds(ooff, CHUNK)],
                osems[c & 1],
            )
        out_descs[0].wait()
        out_descs[1].wait()


def kernel(x):
    flat = _get_pairs_kernel()(
        x.reshape(-1), jnp.asarray(_HEADERS), jnp.asarray(_ITEMS)
    )
    return flat.reshape(NB, 2, 128).swapaxes(1, 2).reshape(NPAIRS, 2)
